# baseline (speedup 1.0000x reference)
import jax
import jax.numpy as jnp
from jax import lax
from jax.experimental import pallas as pl
from jax.experimental.pallas import tpu as pltpu
from jax.experimental.pallas import tpu_sc as plsc

N = 10000
E = 160000
D = 128
S = 3

NC = 2
NS = 16
DH = D // NC
G = 128
EPW = 10240
K = EPW // G
P = NS * EPW
Np = 10112
ROWS_PER_SUB = Np // NS

_f32 = jnp.float32
_i32 = jnp.int32


def _proj_body(h_ref, asrc_ref, adst_ref, as_out, ad_out):
    h = h_ref[...]
    dn = (((1,), (1,)), ((), ()))
    as_out[...] = lax.dot_general(asrc_ref[...], h, dn,
                                  preferred_element_type=_f32)
    ad_out[...] = lax.dot_general(adst_ref[...], h, dn,
                                  preferred_element_type=_f32)


def _proj(h, att_src, att_dst):
    return pl.pallas_call(
        _proj_body,
        out_shape=(jax.ShapeDtypeStruct((S, N), _f32),
                   jax.ShapeDtypeStruct((S, N), _f32)),
    )(h, att_src, att_dst)


def _sc_edges_body(h0_hbm, h1_hbm, as0, as1, as2, ad0, ad1, ad2,
                   src0, src1, src2, dst0, dst1, dst2,
                   accp_hbm, denp_hbm,
                   as_v, ad_v, src_v, dst_v, rows_v, rows2_v,
                   srcidx_v, dstidx_v, ex_v, exrow_v,
                   srcidx2_v, dstidx2_v, ex2_v, exrow2_v,
                   zb64, zb16, acc_sh, den_sh, sem, sem2,
                   ssem, ssem2, dsem, dsem2):
    cid = lax.axis_index("c")
    sid = lax.axis_index("s")
    ebase = sid * EPW
    rbase = sid * ROWS_PER_SUB

    as_list = [as0, as1, as2]
    ad_list = [ad0, ad1, ad2]
    src_list = [src0, src1, src2]
    dst_list = [dst0, dst1, dst2]

    iota16 = lax.iota(_i32, 16)
    zeros16i = jnp.zeros((16,), _i32)
    zeros16f = jnp.zeros((16,), _f32)

    @pl.loop(0, G)
    def _(r):
        for j in range(DH // 16):
            zb64[r, pl.ds(j * 16, 16)] = zeros16f

    @pl.loop(0, ROWS_PER_SUB)
    def _(r):
        zb16[r, pl.ds(0, 16)] = zeros16f

    for s in range(S):
        for t in range(0, ROWS_PER_SUB, G):
            nr = min(G, ROWS_PER_SUB - t)
            pltpu.sync_copy(zb64.at[pl.ds(0, nr)],
                            acc_sh.at[pl.ds(rbase + t, nr)])
        pltpu.sync_copy(zb16, den_sh.at[pl.ds(rbase, ROWS_PER_SUB)])

        pltpu.sync_copy(as_list[s], as_v)
        pltpu.sync_copy(ad_list[s], ad_v)
        pltpu.sync_copy(src_list[s].at[pl.ds(ebase, EPW)], src_v)
        pltpu.sync_copy(dst_list[s].at[pl.ds(ebase, EPW)], dst_v)

        plsc.subcore_barrier()

        def do_scalar(k, sidx, didx, exb, exrow):
            base = k * G
            for g in range(G // 16):
                off = base + g * 16
                sv = src_v[pl.ds(off, 16)]
                dv = dst_v[pl.ds(off, 16)]
                sidx[pl.ds(g * 16, 16)] = sv
                didx[pl.ds(g * 16, 16)] = dv
                al = (plsc.load_gather(as_v, [sv]) +
                      plsc.load_gather(ad_v, [dv]))
                al = jnp.where(al >= 0, al, al * _f32(0.01))
                exv = jnp.exp(al)
                gid = ebase + off + iota16
                exv = jnp.where(gid < E, exv, _f32(0.0))
                exb[pl.ds(g * 16, 16)] = exv
                plsc.store_scatter(exrow, [g * 16 + iota16, zeros16i], exv)

        def do_issue(sidx, rows, gsem):
            @pl.when(cid == 0)
            def _():
                pltpu.async_copy(h0_hbm.at[sidx], rows, gsem)

            @pl.when(cid == 1)
            def _():
                pltpu.async_copy(h1_hbm.at[sidx], rows, gsem)

        def do_wait(sidx, rows, gsem):
            pltpu.make_async_copy(h0_hbm.at[sidx], rows, gsem).wait()

        def do_scale(rows, exb):
            @pl.loop(0, G)
            def _(r):
                wv = plsc.load_gather(exb, [jnp.broadcast_to(r, (16,))])
                for j in range(DH // 16):
                    rows[r, pl.ds(j * 16, 16)] = (
                        rows[r, pl.ds(j * 16, 16)] * wv)

        def issue_scat(rows, didx, exrow, ssem, dsem):
            pltpu.async_copy(rows, acc_sh.at[didx], ssem, add=True)

            @pl.when(cid == 0)
            def _():
                pltpu.async_copy(exrow, den_sh.at[didx], dsem, add=True)

        def wait_scat(rows, didx, exrow, ssem, dsem):
            pltpu.make_async_copy(rows, acc_sh.at[didx], ssem).wait()

            @pl.when(cid == 0)
            def _():
                pltpu.make_async_copy(exrow, den_sh.at[didx], dsem).wait()

        bufA = (srcidx_v, dstidx_v, ex_v, exrow_v, rows_v, sem, ssem, dsem)
        bufB = (srcidx2_v, dstidx2_v, ex2_v, exrow2_v, rows2_v, sem2,
                ssem2, dsem2)

        def stage_next(k, buf):
            sidx, didx, exb, exrow, rows, gsem, _ss, _ds = buf
            do_scalar(k, sidx, didx, exb, exrow)
            do_issue(sidx, rows, gsem)

        def scale_and_scat(buf):
            sidx, didx, exb, exrow, rows, gsem, ss, ds = buf
            do_wait(sidx, rows, gsem)
            do_scale(rows, exb)
            issue_scat(rows, didx, exrow, ss, ds)

        def drain_scat(buf):
            sidx, didx, exb, exrow, rows, gsem, ss, ds = buf
            wait_scat(rows, didx, exrow, ss, ds)

        stage_next(0, bufA)
        stage_next(1, bufB)

        @pl.loop(0, K // 2 - 1)
        def _(t):
            k = 2 * t
            scale_and_scat(bufA)
            scale_and_scat(bufB)
            drain_scat(bufA)
            stage_next(k + 2, bufA)
            drain_scat(bufB)
            stage_next(k + 3, bufB)

        scale_and_scat(bufA)
        scale_and_scat(bufB)
        drain_scat(bufA)
        drain_scat(bufB)

        plsc.subcore_barrier()

        pltpu.sync_copy(acc_sh.at[pl.ds(rbase, ROWS_PER_SUB)],
                        accp_hbm.at[s, cid, pl.ds(rbase, ROWS_PER_SUB)])

        @pl.when(cid == 0)
        def _():
            pltpu.sync_copy(den_sh.at[pl.ds(rbase, ROWS_PER_SUB)],
                            denp_hbm.at[s, pl.ds(rbase, ROWS_PER_SUB)])


def _sc_edges(h0, h1, as_l, ad_l, src_l, dst_l):
    mesh = plsc.VectorSubcoreMesh(core_axis_name="c", subcore_axis_name="s")
    fn = pl.kernel(
        _sc_edges_body,
        out_type=(jax.ShapeDtypeStruct((S, NC, Np, DH), _f32),
                  jax.ShapeDtypeStruct((S, Np, 16), _f32)),
        mesh=mesh,
        compiler_params=pltpu.CompilerParams(needs_layout_passes=False,
                                             use_tc_tiling_on_sc=False),
        scratch_types=[
            pltpu.VMEM((N,), _f32),
            pltpu.VMEM((N,), _f32),
            pltpu.VMEM((EPW,), _i32),
            pltpu.VMEM((EPW,), _i32),
            pltpu.VMEM((G, DH), _f32),
            pltpu.VMEM((G, DH), _f32),
            pltpu.VMEM((G,), _i32),
            pltpu.VMEM((G,), _i32),
            pltpu.VMEM((G,), _f32),
            pltpu.VMEM((G, 16), _f32),
            pltpu.VMEM((G,), _i32),
            pltpu.VMEM((G,), _i32),
            pltpu.VMEM((G,), _f32),
            pltpu.VMEM((G, 16), _f32),
            pltpu.VMEM((G, DH), _f32),
            pltpu.VMEM((ROWS_PER_SUB, 16), _f32),
            pltpu.VMEM_SHARED((Np, DH), _f32),
            pltpu.VMEM_SHARED((Np, 16), _f32),
            pltpu.SemaphoreType.DMA,
            pltpu.SemaphoreType.DMA,
            pltpu.SemaphoreType.DMA,
            pltpu.SemaphoreType.DMA,
            pltpu.SemaphoreType.DMA,
            pltpu.SemaphoreType.DMA,
        ],
    )
    return fn(h0, h1, *as_l, *ad_l, *src_l, *dst_l)


def _fin1_body(accp_ref, denp_ref, bias_ref, w_ref, b_ref, att_ref,
               emb_out, spm_out):
    s = pl.program_id(0)
    acc = accp_ref[...]
    num = jnp.concatenate([acc[0, 0], acc[0, 1]], axis=1)
    d = denp_ref[..., 0][0]
    rows = lax.broadcasted_iota(_i32, (S, D), 0)
    bias = jnp.sum(jnp.where(rows == s, bias_ref[...], _f32(0.0)),
                   axis=0, keepdims=True)
    emb = num / (d[:, None] + _f32(1e-16)) + bias
    emb_out[0] = emb
    proj = jnp.tanh(
        lax.dot_general(emb, w_ref[...], (((1,), (0,)), ((), ())),
                        preferred_element_type=_f32) + b_ref[...][None, :])
    sp = lax.dot_general(proj, att_ref[...], (((1,), (0,)), ((), ())),
                         preferred_element_type=_f32)
    valid = lax.broadcasted_iota(_i32, (Np,), 0) < N
    val = jnp.sum(jnp.where(valid, sp, _f32(0.0))) / _f32(N)
    srow = lax.broadcasted_iota(_i32, (S, 1), 0)
    spm_out[...] = jnp.where(srow == s, val, spm_out[...])


def _fin1(accp, denp, gat_bias, agg_W, agg_b, agg_att):
    return pl.pallas_call(
        _fin1_body,
        grid=(S,),
        in_specs=[
            pl.BlockSpec((1, NC, Np, DH), lambda s: (s, 0, 0, 0)),
            pl.BlockSpec((1, Np, 16), lambda s: (s, 0, 0)),
            pl.BlockSpec((S, D), lambda s: (0, 0)),
            pl.BlockSpec((D, D), lambda s: (0, 0)),
            pl.BlockSpec((D,), lambda s: (0,)),
            pl.BlockSpec((D,), lambda s: (0,)),
        ],
        out_specs=[
            pl.BlockSpec((1, Np, D), lambda s: (s, 0, 0)),
            pl.BlockSpec((S, 1), lambda s: (0, 0)),
        ],
        out_shape=(jax.ShapeDtypeStruct((S, Np, D), _f32),
                   jax.ShapeDtypeStruct((S, 1), _f32)),
    )(accp, denp, gat_bias, agg_W, agg_b, agg_att)


def _fin2_body(emb_ref, spm_ref, z_out):
    beta = jax.nn.softmax(spm_ref[...][:, 0])
    emb = emb_ref[...]
    z_out[...] = jnp.sum(beta[:, None, None] * emb, axis=0)


def _fin2(emb, spm):
    return pl.pallas_call(
        _fin2_body,
        out_shape=jax.ShapeDtypeStruct((Np, D), _f32),
    )(emb, spm)


def kernel(h, att_src, att_dst, gat_bias, agg_W, agg_b, agg_att,
           edge_index0, edge_index1, edge_index2):
    eis = [edge_index0, edge_index1, edge_index2]
    src_l = [jnp.pad(ei[0], (0, P - E)) for ei in eis]
    dst_l = [jnp.pad(ei[1], (0, P - E)) for ei in eis]
    h0 = h[:, :DH]
    h1 = h[:, DH:]
    as_all, ad_all = _proj(h, att_src, att_dst)
    as_l = [as_all[i] for i in range(S)]
    ad_l = [ad_all[i] for i in range(S)]
    accp, denp = _sc_edges(h0, h1, as_l, ad_l, src_l, dst_l)
    emb, spm = _fin1(accp, denp, gat_bias, agg_W, agg_b, agg_att)
    z = _fin2(emb, spm)
    return z[:N]


# speedup vs baseline: 1.1123x; 1.1123x over previous
import jax
import jax.numpy as jnp
from jax import lax
from jax.experimental import pallas as pl
from jax.experimental.pallas import tpu as pltpu
from jax.experimental.pallas import tpu_sc as plsc

N = 10000
E = 160000
D = 128
S = 3

NC = 2
NS = 16
DH = D // NC
G = 128
EPW = 10240
K = EPW // G
P = NS * EPW
Np = 10112
ROWS_PER_SUB = Np // NS

_f32 = jnp.float32
_i32 = jnp.int32


def _proj_body(h_ref, asrc_ref, adst_ref, as_out, ad_out):
    h = h_ref[...]
    dn = (((1,), (1,)), ((), ()))
    as_out[...] = lax.dot_general(asrc_ref[...], h, dn,
                                  preferred_element_type=_f32)
    ad_out[...] = lax.dot_general(adst_ref[...], h, dn,
                                  preferred_element_type=_f32)


def _proj(h, att_src, att_dst):
    return pl.pallas_call(
        _proj_body,
        out_shape=(jax.ShapeDtypeStruct((S, N), _f32),
                   jax.ShapeDtypeStruct((S, N), _f32)),
    )(h, att_src, att_dst)


def _sc_edges_body(h0_hbm, h1_hbm, as0, as1, as2, ad0, ad1, ad2,
                   src0, src1, src2, dst0, dst1, dst2,
                   accp_hbm, denp_hbm,
                   as_v, ad_v, src_v, dst_v, rows_v, rows2_v,
                   srcidx_v, dstidx_v, ex_v, exrow_v,
                   srcidx2_v, dstidx2_v, ex2_v, exrow2_v,
                   zb64, zb16, acc_sh, den_sh, sem, sem2):
    cid = lax.axis_index("c")
    sid = lax.axis_index("s")
    ebase = sid * EPW
    rbase = sid * ROWS_PER_SUB

    as_list = [as0, as1, as2]
    ad_list = [ad0, ad1, ad2]
    src_list = [src0, src1, src2]
    dst_list = [dst0, dst1, dst2]

    iota16 = lax.iota(_i32, 16)
    zeros16i = jnp.zeros((16,), _i32)
    zeros16f = jnp.zeros((16,), _f32)

    @pl.loop(0, G)
    def _(r):
        for j in range(DH // 16):
            zb64[r, pl.ds(j * 16, 16)] = zeros16f

    @pl.loop(0, ROWS_PER_SUB)
    def _(r):
        zb16[r, pl.ds(0, 16)] = zeros16f

    for s in range(S):
        for t in range(0, ROWS_PER_SUB, G):
            nr = min(G, ROWS_PER_SUB - t)
            pltpu.sync_copy(zb64.at[pl.ds(0, nr)],
                            acc_sh.at[pl.ds(rbase + t, nr)])
        pltpu.sync_copy(zb16, den_sh.at[pl.ds(rbase, ROWS_PER_SUB)])

        pltpu.sync_copy(as_list[s], as_v)
        pltpu.sync_copy(ad_list[s], ad_v)
        pltpu.sync_copy(src_list[s].at[pl.ds(ebase, EPW)], src_v)
        pltpu.sync_copy(dst_list[s].at[pl.ds(ebase, EPW)], dst_v)

        plsc.subcore_barrier()

        def do_scalar(k, sidx, didx, exb, exrow):
            base = k * G
            for g in range(G // 16):
                off = base + g * 16
                sv = src_v[pl.ds(off, 16)]
                dv = dst_v[pl.ds(off, 16)]
                sidx[pl.ds(g * 16, 16)] = sv
                didx[pl.ds(g * 16, 16)] = dv
                al = (plsc.load_gather(as_v, [sv]) +
                      plsc.load_gather(ad_v, [dv]))
                al = jnp.where(al >= 0, al, al * _f32(0.01))
                exv = jnp.exp(al)
                gid = ebase + off + iota16
                exv = jnp.where(gid < E, exv, _f32(0.0))
                exb[pl.ds(g * 16, 16)] = exv
                plsc.store_scatter(exrow, [g * 16 + iota16, zeros16i], exv)

        def do_issue(sidx, rows, gsem):
            @pl.when(cid == 0)
            def _():
                pltpu.async_copy(h0_hbm.at[sidx], rows, gsem)

            @pl.when(cid == 1)
            def _():
                pltpu.async_copy(h1_hbm.at[sidx], rows, gsem)

        def do_wait(sidx, rows, gsem):
            pltpu.make_async_copy(h0_hbm.at[sidx], rows, gsem).wait()

        def do_proc(rows, exb, didx, exrow):
            @pl.loop(0, G, unroll=4)
            def _(r):
                wv = plsc.load_gather(exb, [jnp.broadcast_to(r, (16,))])
                for j in range(DH // 16):
                    rows[r, pl.ds(j * 16, 16)] = (
                        rows[r, pl.ds(j * 16, 16)] * wv)

            pltpu.sync_copy(rows, acc_sh.at[didx], add=True)

            @pl.when(cid == 0)
            def _():
                pltpu.sync_copy(exrow, den_sh.at[didx], add=True)

        bufA = (srcidx_v, dstidx_v, ex_v, exrow_v)
        bufB = (srcidx2_v, dstidx2_v, ex2_v, exrow2_v)

        def stage_next(k, buf, rows, gsem):
            sidx, didx, exb, exrow = buf
            do_scalar(k, sidx, didx, exb, exrow)
            do_issue(sidx, rows, gsem)

        def finish(buf, rows, gsem):
            sidx, didx, exb, exrow = buf
            do_wait(sidx, rows, gsem)
            do_proc(rows, exb, didx, exrow)

        stage_next(0, bufA, rows_v, sem)

        @pl.loop(0, (K - 2) // 2)
        def _(t):
            k = 2 * t
            stage_next(k + 1, bufB, rows2_v, sem2)
            finish(bufA, rows_v, sem)
            stage_next(k + 2, bufA, rows_v, sem)
            finish(bufB, rows2_v, sem2)

        stage_next(K - 1, bufB, rows2_v, sem2)
        finish(bufA, rows_v, sem)
        finish(bufB, rows2_v, sem2)

        plsc.subcore_barrier()

        pltpu.sync_copy(acc_sh.at[pl.ds(rbase, ROWS_PER_SUB)],
                        accp_hbm.at[s, cid, pl.ds(rbase, ROWS_PER_SUB)])

        @pl.when(cid == 0)
        def _():
            pltpu.sync_copy(den_sh.at[pl.ds(rbase, ROWS_PER_SUB)],
                            denp_hbm.at[s, pl.ds(rbase, ROWS_PER_SUB)])


def _sc_edges(h0, h1, as_l, ad_l, src_l, dst_l):
    mesh = plsc.VectorSubcoreMesh(core_axis_name="c", subcore_axis_name="s")
    fn = pl.kernel(
        _sc_edges_body,
        out_type=(jax.ShapeDtypeStruct((S, NC, Np, DH), _f32),
                  jax.ShapeDtypeStruct((S, Np, 16), _f32)),
        mesh=mesh,
        compiler_params=pltpu.CompilerParams(needs_layout_passes=False,
                                             use_tc_tiling_on_sc=False),
        scratch_types=[
            pltpu.VMEM((N,), _f32),
            pltpu.VMEM((N,), _f32),
            pltpu.VMEM((EPW,), _i32),
            pltpu.VMEM((EPW,), _i32),
            pltpu.VMEM((G, DH), _f32),
            pltpu.VMEM((G, DH), _f32),
            pltpu.VMEM((G,), _i32),
            pltpu.VMEM((G,), _i32),
            pltpu.VMEM((G,), _f32),
            pltpu.VMEM((G, 16), _f32),
            pltpu.VMEM((G,), _i32),
            pltpu.VMEM((G,), _i32),
            pltpu.VMEM((G,), _f32),
            pltpu.VMEM((G, 16), _f32),
            pltpu.VMEM((G, DH), _f32),
            pltpu.VMEM((ROWS_PER_SUB, 16), _f32),
            pltpu.VMEM_SHARED((Np, DH), _f32),
            pltpu.VMEM_SHARED((Np, 16), _f32),
            pltpu.SemaphoreType.DMA,
            pltpu.SemaphoreType.DMA,
        ],
    )
    return fn(h0, h1, *as_l, *ad_l, *src_l, *dst_l)


def _fin1_body(accp_ref, denp_ref, bias_ref, w_ref, b_ref, att_ref,
               emb_out, spm_out):
    s = pl.program_id(0)
    acc = accp_ref[...]
    num = jnp.concatenate([acc[0, 0], acc[0, 1]], axis=1)
    d = denp_ref[..., 0][0]
    rows = lax.broadcasted_iota(_i32, (S, D), 0)
    bias = jnp.sum(jnp.where(rows == s, bias_ref[...], _f32(0.0)),
                   axis=0, keepdims=True)
    emb = num / (d[:, None] + _f32(1e-16)) + bias
    emb_out[0] = emb
    proj = jnp.tanh(
        lax.dot_general(emb, w_ref[...], (((1,), (0,)), ((), ())),
                        preferred_element_type=_f32) + b_ref[...][None, :])
    sp = lax.dot_general(proj, att_ref[...], (((1,), (0,)), ((), ())),
                         preferred_element_type=_f32)
    valid = lax.broadcasted_iota(_i32, (Np,), 0) < N
    val = jnp.sum(jnp.where(valid, sp, _f32(0.0))) / _f32(N)
    srow = lax.broadcasted_iota(_i32, (S, 1), 0)
    spm_out[...] = jnp.where(srow == s, val, spm_out[...])


def _fin1(accp, denp, gat_bias, agg_W, agg_b, agg_att):
    return pl.pallas_call(
        _fin1_body,
        grid=(S,),
        in_specs=[
            pl.BlockSpec((1, NC, Np, DH), lambda s: (s, 0, 0, 0)),
            pl.BlockSpec((1, Np, 16), lambda s: (s, 0, 0)),
            pl.BlockSpec((S, D), lambda s: (0, 0)),
            pl.BlockSpec((D, D), lambda s: (0, 0)),
            pl.BlockSpec((D,), lambda s: (0,)),
            pl.BlockSpec((D,), lambda s: (0,)),
        ],
        out_specs=[
            pl.BlockSpec((1, Np, D), lambda s: (s, 0, 0)),
            pl.BlockSpec((S, 1), lambda s: (0, 0)),
        ],
        out_shape=(jax.ShapeDtypeStruct((S, Np, D), _f32),
                   jax.ShapeDtypeStruct((S, 1), _f32)),
    )(accp, denp, gat_bias, agg_W, agg_b, agg_att)


def _fin2_body(emb_ref, spm_ref, z_out):
    beta = jax.nn.softmax(spm_ref[...][:, 0])
    emb = emb_ref[...]
    z_out[...] = jnp.sum(beta[:, None, None] * emb, axis=0)


def _fin2(emb, spm):
    return pl.pallas_call(
        _fin2_body,
        out_shape=jax.ShapeDtypeStruct((Np, D), _f32),
    )(emb, spm)


def kernel(h, att_src, att_dst, gat_bias, agg_W, agg_b, agg_att,
           edge_index0, edge_index1, edge_index2):
    eis = [edge_index0, edge_index1, edge_index2]
    src_l = [jnp.pad(ei[0], (0, P - E)) for ei in eis]
    dst_l = [jnp.pad(ei[1], (0, P - E)) for ei in eis]
    h0 = h[:, :DH]
    h1 = h[:, DH:]
    as_all, ad_all = _proj(h, att_src, att_dst)
    as_l = [as_all[i] for i in range(S)]
    ad_l = [ad_all[i] for i in range(S)]
    accp, denp = _sc_edges(h0, h1, as_l, ad_l, src_l, dst_l)
    emb, spm = _fin1(accp, denp, gat_bias, agg_W, agg_b, agg_att)
    z = _fin2(emb, spm)
    return z[:N]
